# baseline (speedup 1.0000x reference)
import jax
import jax.numpy as jnp
from jax import lax
from jax.experimental import pallas as pl
from jax.experimental.pallas import tpu as pltpu
from jax.experimental.pallas import tpu_sc as plsc

N = 10000
E = 160000
D = 256

NC = 2
NS = 16
NW = NC * NS
L = 16
NRG = 4
PIB = lax.GatherScatterMode.PROMISE_IN_BOUNDS


def _ceil(a, b):
  return (a + b - 1) // b


def _cfg(n, e, d, bk):
  rpw = max(8, (n // NW) & ~7)
  last = n - (NW - 1) * rpw
  rng = 8 * rpw
  rng3 = n - 3 * rng
  eps = e // NW
  epsp = _ceil(eps, L) * L
  dummy = max(rpw, last)
  accr = _ceil(dummy + 1, 8) * 8
  sh2 = max(accr - 1, 1).bit_length()
  sh1 = max(rng, rng3).bit_length()
  assert last % 8 == 0 and rpw % 8 == 0 and bk % L == 0 and bk <= 128
  cap1 = 16 + epsp + L
  cap2 = 16 + epsp + 4 * bk
  assert cap1 % 16 == 0 and cap2 % 16 == 0
  return dict(n=n, e=e, d=d, bk=bk, rpw=rpw, last=last, rng=rng, rng3=rng3,
              eps=eps, epsp=epsp, dummy=dummy, accr=accr, sh1=sh1, sh2=sh2,
              cap1=cap1, cap2=cap2)


CFG = _cfg(N, E, D, bk=64)


def _mesh():
  return plsc.VectorSubcoreMesh(core_axis_name="c", subcore_axis_name="s",
                                num_cores=NC, num_subcores=NS)


def _wid():
  return lax.axis_index("c") * NS + lax.axis_index("s")


def _insert_lanes(bucket, hdr, iota, state, vals, oks):
  pending, pendprev, pos, previdx = state
  for k in range(L):
    vk = vals[k]
    okk = oks[k]
    pe = (pos & (L - 1)) * okk + (okk - 1)
    pending = jnp.where(iota == pe, vk, pending)
    pos = pos + okk
    cross = (okk == 1) & ((pos & (L - 1)) == 0)
    pendprev = jnp.where(cross, pending, pendprev)
    previdx = jnp.where(cross, (pos >> 4) - 1, previdx)
  return pending, pendprev, pos, previdx


def _store_blocks(bucket, hdr, state):
  pending, pendprev, pos, previdx = state
  bucket[pl.ds(hdr + previdx * L, L)] = pendprev
  bucket[pl.ds(hdr + ((pos >> 4) << 4), L)] = pending


def _finish_bucket(bucket, hdr, iota, pending, pos, padv, npad):
  rem = pos & (L - 1)
  tail = jnp.where(iota < rem, pending, padv)
  b0 = hdr + pos - rem
  bucket[pl.ds(b0, L)] = tail
  for u in range(1, npad + 1):
    bucket[pl.ds(b0 + u * L, L)] = padv
  bucket[pl.ds(0, L)] = jnp.zeros((L,), jnp.int32) + pos


def _make_bin1(cfg):
  eps, epsp = cfg["eps"], cfg["epsp"]
  rng, rng3, sh1 = cfg["rng"], cfg["rng3"], cfg["sh1"]

  def body(src_hbm, dst_hbm, out_hbm, srcc, dstc, b0, b1, b2, b3):
    w = _wid()
    iota = lax.iota(jnp.int32, L)
    padv = jnp.full((L,), jnp.int32((1 << sh1) - 1), jnp.int32)

    e0 = pl.multiple_of(w * eps, 8)
    pltpu.sync_copy(src_hbm.at[pl.ds(e0, eps)], srcc.at[pl.ds(0, eps)])
    pltpu.sync_copy(dst_hbm.at[pl.ds(e0, eps)], dstc.at[pl.ds(0, eps)])
    rem = eps % L
    t0 = eps - rem
    if rem:
      dv = dstc[pl.ds(t0, L)]
      dstc[pl.ds(t0, L)] = jnp.where(iota < rem, dv,
                                     jnp.int32(1 << 30))
      sv = srcc[pl.ds(t0, L)]
      srcc[pl.ds(t0, L)] = jnp.where(iota < rem, sv, 0)
      t0 += L
    for t in range(t0 // L, epsp // L):
      srcc[pl.ds(t * L, L)] = jnp.zeros((L,), jnp.int32)
      dstc[pl.ds(t * L, L)] = jnp.full((L,), jnp.int32(1 << 30), jnp.int32)

    bufs = (b0, b1, b2, b3)
    sizes = (rng, rng, rng, rng3)

    def _grp(t, carry):
      states = [tuple(carry[4 * r:4 * r + 4]) for r in range(NRG)]
      dv = dstc[pl.ds(t * L, L)]
      sv = srcc[pl.ds(t * L, L)]
      out = []
      for r in range(NRG):
        lb = dv - r * rng
        oki = 1 - lax.shift_right_logical(lb | (sizes[r] - 1 - lb), 31)
        vals = (sv << sh1) | (lb * oki)
        st = _insert_lanes(bufs[r], 16, iota, states[r], vals, oki)
        _store_blocks(bufs[r], 16, st)
        out.extend(st)
      return tuple(out)

    zv = jnp.zeros((L,), jnp.int32)
    init = ()
    for _ in range(NRG):
      init = init + (zv, zv, jnp.int32(0), jnp.int32(0))
    carry = lax.fori_loop(0, epsp // L, _grp, init)
    for r in range(NRG):
      st = tuple(carry[4 * r:4 * r + 4])
      _store_blocks(bufs[r], 16, st)
      _finish_bucket(bufs[r], 16, iota, st[0], st[2], padv, 0)
      pltpu.sync_copy(bufs[r], out_hbm.at[w, r])

  return body


def _bin1(src, dst, cfg=CFG, *, interpret=False):
  f = pl.kernel(
      _make_bin1(cfg),
      out_type=jax.ShapeDtypeStruct((NW, NRG, cfg["cap1"]), jnp.int32),
      mesh=_mesh(),
      scratch_types=[
          pltpu.VMEM((cfg["epsp"],), jnp.int32),
          pltpu.VMEM((cfg["epsp"],), jnp.int32),
          pltpu.VMEM((cfg["cap1"],), jnp.int32),
          pltpu.VMEM((cfg["cap1"],), jnp.int32),
          pltpu.VMEM((cfg["cap1"],), jnp.int32),
          pltpu.VMEM((cfg["cap1"],), jnp.int32),
      ],
      interpret=interpret,
  )
  return f(src, dst)


def _make_bin2(cfg):
  rpw, last, rng, sh1, sh2, bk = (cfg["rpw"], cfg["last"], cfg["rng"],
                                  cfg["sh1"], cfg["sh2"], cfg["bk"])
  dummy = cfg["dummy"]

  def body(slots_hbm, out_hbm, segbuf, bucket):
    w = _wid()
    iota = lax.iota(jnp.int32, L)
    padv = jnp.full((L,), jnp.int32(dummy), jnp.int32)
    r = w >> 3
    rbase = r * rng
    base = pl.multiple_of(w * rpw, 8)
    bound = jnp.where(w == NW - 1, last, rpw)
    mask1 = (1 << sh1) - 1

    def _seg(i, _):
      pltpu.sync_copy(slots_hbm.at[i, r], segbuf)
      cnt = segbuf[pl.ds(0, L)][0]

      def _grp(t, carry):
        p = segbuf[pl.ds(16 + t * L, L)]
        gdst = rbase + (p & mask1)
        lv = gdst - base
        oki = 1 - lax.shift_right_logical(lv | (bound - 1 - lv), 31)
        src = lax.shift_right_logical(p, sh1)
        vals = (src << sh2) | (lv * oki + jnp.int32(dummy) * (1 - oki))
        st = _insert_lanes(bucket, 16, iota, tuple(carry), vals, oki)
        _store_blocks(bucket, 16, st)
        return st

      zv = jnp.zeros((L,), jnp.int32)
      st = lax.fori_loop(0, (cnt + L - 1) // L, _grp,
                         (zv, zv, jnp.int32(0), jnp.int32(0)))
      _store_blocks(bucket, 16, st)
      _finish_bucket(bucket, 16, iota, st[0], st[2], padv,
                     4 * bk // L - 1)
      pltpu.sync_copy(bucket, out_hbm.at[w, i])
      return 0
    lax.fori_loop(0, NW, _seg, 0)

  return body


def _bin2(slots, cfg=CFG, *, interpret=False):
  f = pl.kernel(
      _make_bin2(cfg),
      out_type=jax.ShapeDtypeStruct((NW, NW, cfg["cap2"]), jnp.int32),
      mesh=_mesh(),
      scratch_types=[
          pltpu.VMEM((cfg["cap1"],), jnp.int32),
          pltpu.VMEM((cfg["cap2"],), jnp.int32),
      ],
      interpret=interpret,
  )
  return f(slots)


def _make_acc(cfg, dh):
  bk = cfg["bk"]
  rpw, last, accr, sh2 = cfg["rpw"], cfg["last"], cfg["accr"], cfg["sh2"]
  ncg = dh // L
  mask2 = (1 << sh2) - 1

  def body(xh_hbm, slots_hbm, out_hbm, segbuf,
           sidx0, sidx1, sidx2, sidx3, rows0, rows1, rows2, rows3,
           acc, cnt, sem0, sem1, sem2, sem3):
    w = _wid()
    base = pl.multiple_of(w * rpw, 8)
    zf = jnp.zeros((L,), jnp.float32)
    onef = jnp.ones((L,), jnp.float32)
    sidxs = (sidx0, sidx1, sidx2, sidx3)
    rowss = (rows0, rows1, rows2, rows3)
    sems = (sem0, sem1, sem2, sem3)

    def _zr(rr, _):
      for g in range(ncg):
        acc[rr, pl.ds(g * L, L)] = zf
      cnt[rr, :] = zf
      return 0
    lax.fori_loop(0, accr, _zr, 0)

    def _seg(i, _):
      pltpu.sync_copy(slots_hbm.at[w, i], segbuf)
      ecnt = segbuf[pl.ds(0, L)][0]

      def _quad(q, _):
        descs = []
        for j in range(4):
          b0 = 16 + (q * 4 + j) * bk
          for u in range(bk // L):
            p = segbuf[pl.ds(b0 + u * L, L)]
            sidxs[j][pl.ds(u * L, L)] = lax.shift_right_logical(p, sh2)
          descs.append(pltpu.async_copy(xh_hbm.at[sidxs[j]], rowss[j],
                                        sems[j]))
        for j in range(4):
          descs[j].wait()
          b0 = 16 + (q * 4 + j) * bk
          rows_v = rowss[j]

          def _rmw(g, _, b0=b0, rows_v=rows_v):
            p = segbuf[pl.ds(b0 + g * L, L)]
            locs = p & mask2
            for k in range(L):
              rr = locs[k]
              sr = g * L + k
              for gc in range(ncg):
                acc[rr, pl.ds(gc * L, L)] = (
                    acc[rr, pl.ds(gc * L, L)] + rows_v[sr, pl.ds(gc * L, L)])
              cnt[rr, :] = cnt[rr, :] + onef
            return 0
          lax.fori_loop(0, bk // L, _rmw, 0)
        return 0
      lax.fori_loop(0, (ecnt + 4 * bk - 1) // (4 * bk), _quad, 0)
      return 0
    lax.fori_loop(0, NW, _seg, 0)

    def _div(rr, _):
      inv = 1.0 / jnp.maximum(cnt[rr, :], 1.0)
      for gc in range(ncg):
        acc[rr, pl.ds(gc * L, L)] = acc[rr, pl.ds(gc * L, L)] * inv
      return 0
    lax.fori_loop(0, last, _div, 0)

    @pl.when(w < NW - 1)
    def _():
      pltpu.sync_copy(acc.at[pl.ds(0, rpw)], out_hbm.at[pl.ds(base, rpw)])
    @pl.when(w == NW - 1)
    def _():
      pltpu.sync_copy(acc.at[pl.ds(0, last)], out_hbm.at[pl.ds(base, last)])

  return body


def _acc_half(xh, slots, cfg=CFG, *, interpret=False):
  dh = xh.shape[1]
  f = pl.kernel(
      _make_acc(cfg, dh),
      out_type=jax.ShapeDtypeStruct((cfg["n"], dh), jnp.float32),
      mesh=_mesh(),
      scratch_types=[
          pltpu.VMEM((cfg["cap2"],), jnp.int32),
          pltpu.VMEM((cfg["bk"],), jnp.int32),
          pltpu.VMEM((cfg["bk"],), jnp.int32),
          pltpu.VMEM((cfg["bk"],), jnp.int32),
          pltpu.VMEM((cfg["bk"],), jnp.int32),
          pltpu.VMEM((cfg["bk"], dh), jnp.float32),
          pltpu.VMEM((cfg["bk"], dh), jnp.float32),
          pltpu.VMEM((cfg["bk"], dh), jnp.float32),
          pltpu.VMEM((cfg["bk"], dh), jnp.float32),
          pltpu.VMEM((cfg["accr"], dh), jnp.float32),
          pltpu.VMEM((cfg["accr"], L), jnp.float32),
          pltpu.SemaphoreType.DMA,
          pltpu.SemaphoreType.DMA,
          pltpu.SemaphoreType.DMA,
          pltpu.SemaphoreType.DMA,
      ],
      interpret=interpret,
  )
  return f(xh, slots)


def _segment_mean_halves(x, src, dst, cfg=CFG, *, interpret=False):
  dh = cfg["d"] // 2
  slots1 = _bin1(src, dst, cfg, interpret=interpret)
  slots2 = _bin2(slots1, cfg, interpret=interpret)
  m0 = _acc_half(x[:, :dh], slots2, cfg, interpret=interpret)
  m1 = _acc_half(x[:, dh:], slots2, cfg, interpret=interpret)
  return m0, m1


def _tc_body(m0_ref, m1_ref, x_ref, wl0_ref, wl1_ref, bl_ref, wr_ref,
             wlin_ref, blin_ref, out_ref):
  h = jnp.dot(m0_ref[...], wl0_ref[...], preferred_element_type=jnp.float32)
  h = h + jnp.dot(m1_ref[...], wl1_ref[...],
                  preferred_element_type=jnp.float32)
  h = h + bl_ref[...]
  h = h + jnp.dot(x_ref[...], wr_ref[...], preferred_element_type=jnp.float32)
  h = jnp.maximum(h, 0.0)
  out_ref[...] = (
      jnp.dot(h, wlin_ref[...], preferred_element_type=jnp.float32)
      + blin_ref[...])


def _dense(m0, m1, x, W_l, b_l, W_r, W_lin, b_lin, *, interpret=False):
  rows = 1000
  dh = D // 2
  grid = (N // rows,)
  half_spec = pl.BlockSpec((rows, dh), lambda i: (i, 0))
  row_spec = pl.BlockSpec((rows, D), lambda i: (i, 0))
  w_spec = pl.BlockSpec((D, D), lambda i: (0, 0))
  wh_spec = pl.BlockSpec((dh, D), lambda i: (0, 0))
  b_spec = pl.BlockSpec((1, D), lambda i: (0, 0))
  return pl.pallas_call(
      _tc_body,
      grid=grid,
      in_specs=[half_spec, half_spec, row_spec, wh_spec, wh_spec, b_spec,
                w_spec, w_spec, b_spec],
      out_specs=row_spec,
      out_shape=jax.ShapeDtypeStruct((N, D), jnp.float32),
      interpret=interpret,
  )(m0, m1, x, W_l[:dh], W_l[dh:], b_l.reshape(1, D), W_r, W_lin,
    b_lin.reshape(1, D))


@jax.jit
def kernel(x, edge_index, W_l, b_l, W_r, W_lin, b_lin):
  src = edge_index[0].astype(jnp.int32)
  dst = edge_index[1].astype(jnp.int32)
  m0, m1 = _segment_mean_halves(x, src, dst)
  return _dense(m0, m1, x, W_l, b_l, W_r, W_lin, b_lin)


# speedup vs baseline: 2.5456x; 2.5456x over previous
import jax
import jax.numpy as jnp
from jax import lax
from jax.experimental import pallas as pl
from jax.experimental.pallas import tpu as pltpu
from jax.experimental.pallas import tpu_sc as plsc

N = 10000
E = 160000
D = 256

NC = 2
NS = 16
NW = NC * NS
L = 16
NRG = 4
PIB = lax.GatherScatterMode.PROMISE_IN_BOUNDS


def _ceil(a, b):
  return (a + b - 1) // b


def _cfg(n, e, d, bk):
  rpw = max(8, (n // NW) & ~7)
  last = n - (NW - 1) * rpw
  rng = 8 * rpw
  rng3 = n - 3 * rng
  eps = e // NW
  epsp = _ceil(eps, L) * L
  dummy = max(rpw, last)
  accr = _ceil(dummy + 1, 8) * 8
  sh2 = max(accr - 1, 1).bit_length()
  sh1 = max(rng, rng3).bit_length()
  assert last % 8 == 0 and rpw % 8 == 0 and bk % L == 0 and bk <= 128
  cap1 = 16 + epsp + L
  cap2 = 16 + epsp + 4 * bk
  assert cap1 % 16 == 0 and cap2 % 16 == 0
  return dict(n=n, e=e, d=d, bk=bk, rpw=rpw, last=last, rng=rng, rng3=rng3,
              eps=eps, epsp=epsp, dummy=dummy, accr=accr, sh1=sh1, sh2=sh2,
              cap1=cap1, cap2=cap2)


CFG = _cfg(N, E, D, bk=64)


def _mesh():
  return plsc.VectorSubcoreMesh(core_axis_name="c", subcore_axis_name="s",
                                num_cores=NC, num_subcores=NS)


def _wid():
  return lax.axis_index("c") * NS + lax.axis_index("s")


def _insert_lanes(bucket, hdr, iota, state, vals, oks):
  pending, pendprev, pos, previdx = state
  for k in range(L):
    vk = vals[k]
    okk = oks[k]
    pe = (pos & (L - 1)) * okk + (okk - 1)
    pending = jnp.where(iota == pe, vk, pending)
    pos = pos + okk
    cross = (okk == 1) & ((pos & (L - 1)) == 0)
    pendprev = jnp.where(cross, pending, pendprev)
    previdx = jnp.where(cross, (pos >> 4) - 1, previdx)
  return pending, pendprev, pos, previdx


def _store_blocks(bucket, hdr, state):
  pending, pendprev, pos, previdx = state
  bucket[pl.ds(hdr + previdx * L, L)] = pendprev
  bucket[pl.ds(hdr + ((pos >> 4) << 4), L)] = pending


def _finish_bucket(bucket, hdr, iota, pending, pos, padv, npad):
  rem = pos & (L - 1)
  tail = jnp.where(iota < rem, pending, padv)
  b0 = hdr + pos - rem
  bucket[pl.ds(b0, L)] = tail
  for u in range(1, npad + 1):
    bucket[pl.ds(b0 + u * L, L)] = padv
  bucket[pl.ds(0, L)] = jnp.zeros((L,), jnp.int32) + pos


def _make_bin1(cfg):
  eps, epsp = cfg["eps"], cfg["epsp"]
  rng, rng3, sh1 = cfg["rng"], cfg["rng3"], cfg["sh1"]

  def body(src_hbm, dst_hbm, out_hbm, srcc, dstc, b0, b1, b2, b3):
    w = _wid()
    iota = lax.iota(jnp.int32, L)
    padv = jnp.full((L,), jnp.int32((1 << sh1) - 1), jnp.int32)

    e0 = pl.multiple_of(w * eps, 8)
    pltpu.sync_copy(src_hbm.at[pl.ds(e0, eps)], srcc.at[pl.ds(0, eps)])
    pltpu.sync_copy(dst_hbm.at[pl.ds(e0, eps)], dstc.at[pl.ds(0, eps)])
    rem = eps % L
    t0 = eps - rem
    if rem:
      dv = dstc[pl.ds(t0, L)]
      dstc[pl.ds(t0, L)] = jnp.where(iota < rem, dv,
                                     jnp.int32(1 << 30))
      sv = srcc[pl.ds(t0, L)]
      srcc[pl.ds(t0, L)] = jnp.where(iota < rem, sv, 0)
      t0 += L
    for t in range(t0 // L, epsp // L):
      srcc[pl.ds(t * L, L)] = jnp.zeros((L,), jnp.int32)
      dstc[pl.ds(t * L, L)] = jnp.full((L,), jnp.int32(1 << 30), jnp.int32)

    bufs = (b0, b1, b2, b3)
    sizes = (rng, rng, rng, rng3)

    def _grp(t, carry):
      states = [tuple(carry[4 * r:4 * r + 4]) for r in range(NRG)]
      dv = dstc[pl.ds(t * L, L)]
      sv = srcc[pl.ds(t * L, L)]
      out = []
      for r in range(NRG):
        lb = dv - r * rng
        oki = 1 - lax.shift_right_logical(lb | (sizes[r] - 1 - lb), 31)
        vals = (sv << sh1) | (lb * oki)
        st = _insert_lanes(bufs[r], 16, iota, states[r], vals, oki)
        _store_blocks(bufs[r], 16, st)
        out.extend(st)
      return tuple(out)

    zv = jnp.zeros((L,), jnp.int32)
    init = ()
    for _ in range(NRG):
      init = init + (zv, zv, jnp.int32(0), jnp.int32(0))
    carry = lax.fori_loop(0, epsp // L, _grp, init)
    for r in range(NRG):
      st = tuple(carry[4 * r:4 * r + 4])
      _store_blocks(bufs[r], 16, st)
      _finish_bucket(bufs[r], 16, iota, st[0], st[2], padv, 0)
      pltpu.sync_copy(bufs[r], out_hbm.at[w, r])

  return body


def _bin1(src, dst, cfg=CFG, *, interpret=False):
  f = pl.kernel(
      _make_bin1(cfg),
      out_type=jax.ShapeDtypeStruct((NW, NRG, cfg["cap1"]), jnp.int32),
      mesh=_mesh(),
      scratch_types=[
          pltpu.VMEM((cfg["epsp"],), jnp.int32),
          pltpu.VMEM((cfg["epsp"],), jnp.int32),
          pltpu.VMEM((cfg["cap1"],), jnp.int32),
          pltpu.VMEM((cfg["cap1"],), jnp.int32),
          pltpu.VMEM((cfg["cap1"],), jnp.int32),
          pltpu.VMEM((cfg["cap1"],), jnp.int32),
      ],
      interpret=interpret,
  )
  return f(src, dst)


def _make_bin2(cfg):
  rpw, last, rng, sh1, sh2, bk = (cfg["rpw"], cfg["last"], cfg["rng"],
                                  cfg["sh1"], cfg["sh2"], cfg["bk"])
  dummy = cfg["dummy"]

  def body(slots_hbm, out_hbm, segbuf, bucket):
    w = _wid()
    iota = lax.iota(jnp.int32, L)
    padv = jnp.full((L,), jnp.int32(dummy), jnp.int32)
    r = w >> 3
    rbase = r * rng
    base = pl.multiple_of(w * rpw, 8)
    bound = jnp.where(w == NW - 1, last, rpw)
    mask1 = (1 << sh1) - 1

    def _seg(i, _):
      pltpu.sync_copy(slots_hbm.at[i, r], segbuf)
      cnt = segbuf[pl.ds(0, L)][0]

      def _grp(t, carry):
        p = segbuf[pl.ds(16 + t * L, L)]
        gdst = rbase + (p & mask1)
        lv = gdst - base
        oki = 1 - lax.shift_right_logical(lv | (bound - 1 - lv), 31)
        src = lax.shift_right_logical(p, sh1)
        vals = (src << sh2) | (lv * oki + jnp.int32(dummy) * (1 - oki))
        st = _insert_lanes(bucket, 16, iota, tuple(carry), vals, oki)
        _store_blocks(bucket, 16, st)
        return st

      zv = jnp.zeros((L,), jnp.int32)
      st = lax.fori_loop(0, (cnt + L - 1) // L, _grp,
                         (zv, zv, jnp.int32(0), jnp.int32(0)))
      _store_blocks(bucket, 16, st)
      _finish_bucket(bucket, 16, iota, st[0], st[2], padv,
                     4 * bk // L - 1)
      pltpu.sync_copy(bucket, out_hbm.at[w, i])
      return 0
    lax.fori_loop(0, NW, _seg, 0)

  return body


def _bin2(slots, cfg=CFG, *, interpret=False):
  f = pl.kernel(
      _make_bin2(cfg),
      out_type=jax.ShapeDtypeStruct((NW, NW, cfg["cap2"]), jnp.int32),
      mesh=_mesh(),
      scratch_types=[
          pltpu.VMEM((cfg["cap1"],), jnp.int32),
          pltpu.VMEM((cfg["cap2"],), jnp.int32),
      ],
      interpret=interpret,
  )
  return f(slots)


def _make_acc(cfg, dh):
  bk = cfg["bk"]
  rpw, last, accr, sh2 = cfg["rpw"], cfg["last"], cfg["accr"], cfg["sh2"]
  ncg = dh // L
  mask2 = (1 << sh2) - 1

  def body(xh_hbm, slots_hbm, out_hbm, segbuf,
           sidx0, sidx1, sidx2, sidx3, rows0, rows1, rows2, rows3,
           acc, cnt, sem0, sem1, sem2, sem3):
    w = _wid()
    base = pl.multiple_of(w * rpw, 8)
    zf = jnp.zeros((L,), jnp.float32)
    onef = jnp.ones((L,), jnp.float32)
    sidxs = (sidx0, sidx1, sidx2, sidx3)
    rowss = (rows0, rows1, rows2, rows3)
    sems = (sem0, sem1, sem2, sem3)

    def _zr(rr, _):
      for g in range(ncg):
        acc[rr, pl.ds(g * L, L)] = zf
      cnt[rr, :] = zf
      return 0
    lax.fori_loop(0, accr, _zr, 0)

    def _seg(i, _):
      pltpu.sync_copy(slots_hbm.at[w, i], segbuf)
      ecnt = segbuf[pl.ds(0, L)][0]

      def _blk(b, _):
        b0 = 16 + b * bk
        for u in range(bk // L):
          p = segbuf[pl.ds(b0 + u * L, L)]
          sidx0[pl.ds(u * L, L)] = lax.shift_right_logical(p, sh2)
        pltpu.async_copy(xh_hbm.at[sidx0], rows0, sem0).wait()

        def _rmw(g, _):
          p = segbuf[pl.ds(b0 + g * L, L)]
          locs = p & mask2
          for k in range(L):
            rr = locs[k]
            sr = g * L + k
            for gc in range(ncg):
              acc[rr, pl.ds(gc * L, L)] = (
                  acc[rr, pl.ds(gc * L, L)] + rows0[sr, pl.ds(gc * L, L)])
            cnt[rr, :] = cnt[rr, :] + onef
          return 0
        lax.fori_loop(0, bk // L, _rmw, 0)
        return 0
      lax.fori_loop(0, (ecnt + bk - 1) // bk, _blk, 0)
      return 0
    lax.fori_loop(0, NW, _seg, 0)

    def _div(rr, _):
      inv = 1.0 / jnp.maximum(cnt[rr, :], 1.0)
      for gc in range(ncg):
        acc[rr, pl.ds(gc * L, L)] = acc[rr, pl.ds(gc * L, L)] * inv
      return 0
    lax.fori_loop(0, last, _div, 0)

    @pl.when(w < NW - 1)
    def _():
      pltpu.sync_copy(acc.at[pl.ds(0, rpw)], out_hbm.at[pl.ds(base, rpw)])
    @pl.when(w == NW - 1)
    def _():
      pltpu.sync_copy(acc.at[pl.ds(0, last)], out_hbm.at[pl.ds(base, last)])

  return body


def _acc_half(xh, slots, cfg=CFG, *, interpret=False):
  dh = xh.shape[1]
  f = pl.kernel(
      _make_acc(cfg, dh),
      out_type=jax.ShapeDtypeStruct((cfg["n"], dh), jnp.float32),
      mesh=_mesh(),
      scratch_types=[
          pltpu.VMEM((cfg["cap2"],), jnp.int32),
          pltpu.VMEM((cfg["bk"],), jnp.int32),
          pltpu.VMEM((cfg["bk"],), jnp.int32),
          pltpu.VMEM((cfg["bk"],), jnp.int32),
          pltpu.VMEM((cfg["bk"],), jnp.int32),
          pltpu.VMEM((cfg["bk"], dh), jnp.float32),
          pltpu.VMEM((cfg["bk"], dh), jnp.float32),
          pltpu.VMEM((cfg["bk"], dh), jnp.float32),
          pltpu.VMEM((cfg["bk"], dh), jnp.float32),
          pltpu.VMEM((cfg["accr"], dh), jnp.float32),
          pltpu.VMEM((cfg["accr"], L), jnp.float32),
          pltpu.SemaphoreType.DMA,
          pltpu.SemaphoreType.DMA,
          pltpu.SemaphoreType.DMA,
          pltpu.SemaphoreType.DMA,
      ],
      interpret=interpret,
  )
  return f(xh, slots)


def _segment_mean_halves(x, src, dst, cfg=CFG, *, interpret=False):
  dh = cfg["d"] // 2
  slots1 = _bin1(src, dst, cfg, interpret=interpret)
  slots2 = _bin2(slots1, cfg, interpret=interpret)
  m0 = _acc_half(x[:, :dh], slots2, cfg, interpret=interpret)
  m1 = _acc_half(x[:, dh:], slots2, cfg, interpret=interpret)
  return m0, m1


def _tc_body(m0_ref, m1_ref, x_ref, wl0_ref, wl1_ref, bl_ref, wr_ref,
             wlin_ref, blin_ref, out_ref):
  h = jnp.dot(m0_ref[...], wl0_ref[...], preferred_element_type=jnp.float32)
  h = h + jnp.dot(m1_ref[...], wl1_ref[...],
                  preferred_element_type=jnp.float32)
  h = h + bl_ref[...]
  h = h + jnp.dot(x_ref[...], wr_ref[...], preferred_element_type=jnp.float32)
  h = jnp.maximum(h, 0.0)
  out_ref[...] = (
      jnp.dot(h, wlin_ref[...], preferred_element_type=jnp.float32)
      + blin_ref[...])


def _dense(m0, m1, x, W_l, b_l, W_r, W_lin, b_lin, *, interpret=False):
  rows = 1000
  dh = D // 2
  grid = (N // rows,)
  half_spec = pl.BlockSpec((rows, dh), lambda i: (i, 0))
  row_spec = pl.BlockSpec((rows, D), lambda i: (i, 0))
  w_spec = pl.BlockSpec((D, D), lambda i: (0, 0))
  wh_spec = pl.BlockSpec((dh, D), lambda i: (0, 0))
  b_spec = pl.BlockSpec((1, D), lambda i: (0, 0))
  return pl.pallas_call(
      _tc_body,
      grid=grid,
      in_specs=[half_spec, half_spec, row_spec, wh_spec, wh_spec, b_spec,
                w_spec, w_spec, b_spec],
      out_specs=row_spec,
      out_shape=jax.ShapeDtypeStruct((N, D), jnp.float32),
      interpret=interpret,
  )(m0, m1, x, W_l[:dh], W_l[dh:], b_l.reshape(1, D), W_r, W_lin,
    b_lin.reshape(1, D))


@jax.jit
def kernel(x, edge_index, W_l, b_l, W_r, W_lin, b_lin):
  src = edge_index[0].astype(jnp.int32)
  dst = edge_index[1].astype(jnp.int32)
  m0, m1 = _segment_mean_halves(x, src, dst)
  return _dense(m0, m1, x, W_l, b_l, W_r, W_lin, b_lin)


# speedup vs baseline: 2.5525x; 1.0027x over previous
import jax
import jax.numpy as jnp
from jax import lax
from jax.experimental import pallas as pl
from jax.experimental.pallas import tpu as pltpu
from jax.experimental.pallas import tpu_sc as plsc

N = 10000
E = 160000
D = 256

NC = 2
NS = 16
NW = NC * NS
L = 16
NRG = 4
PIB = lax.GatherScatterMode.PROMISE_IN_BOUNDS


def _ceil(a, b):
  return (a + b - 1) // b


def _cfg(n, e, d, bk):
  rpw = max(8, (n // NW) & ~7)
  last = n - (NW - 1) * rpw
  rng = 8 * rpw
  rng3 = n - 3 * rng
  eps = e // NW
  epsp = _ceil(eps, L) * L
  dummy = max(rpw, last)
  accr = _ceil(dummy + 1, 8) * 8
  sh2 = max(accr - 1, 1).bit_length()
  sh1 = max(rng, rng3).bit_length()
  assert last % 8 == 0 and rpw % 8 == 0 and bk % L == 0 and bk <= 128
  cap1 = 16 + epsp + L
  cap2 = 16 + epsp + 4 * bk
  assert cap1 % 16 == 0 and cap2 % 16 == 0
  return dict(n=n, e=e, d=d, bk=bk, rpw=rpw, last=last, rng=rng, rng3=rng3,
              eps=eps, epsp=epsp, dummy=dummy, accr=accr, sh1=sh1, sh2=sh2,
              cap1=cap1, cap2=cap2)


CFG = _cfg(N, E, D, bk=64)


def _mesh():
  return plsc.VectorSubcoreMesh(core_axis_name="c", subcore_axis_name="s",
                                num_cores=NC, num_subcores=NS)


def _wid():
  return lax.axis_index("c") * NS + lax.axis_index("s")


def _insert_lanes(bucket, hdr, iota, state, vals, oks):
  pending, pendprev, pos, previdx = state
  for k in range(L):
    vk = vals[k]
    okk = oks[k]
    pe = (pos & (L - 1)) * okk + (okk - 1)
    pending = jnp.where(iota == pe, vk, pending)
    pos = pos + okk
    cross = (okk == 1) & ((pos & (L - 1)) == 0)
    pendprev = jnp.where(cross, pending, pendprev)
    previdx = jnp.where(cross, (pos >> 4) - 1, previdx)
  return pending, pendprev, pos, previdx


def _store_blocks(bucket, hdr, state):
  pending, pendprev, pos, previdx = state
  bucket[pl.ds(hdr + previdx * L, L)] = pendprev
  bucket[pl.ds(hdr + ((pos >> 4) << 4), L)] = pending


def _finish_bucket(bucket, hdr, iota, pending, pos, padv, npad):
  rem = pos & (L - 1)
  tail = jnp.where(iota < rem, pending, padv)
  b0 = hdr + pos - rem
  bucket[pl.ds(b0, L)] = tail
  for u in range(1, npad + 1):
    bucket[pl.ds(b0 + u * L, L)] = padv
  bucket[pl.ds(0, L)] = jnp.zeros((L,), jnp.int32) + pos


def _make_bin1(cfg):
  eps, epsp = cfg["eps"], cfg["epsp"]
  rng, rng3, sh1 = cfg["rng"], cfg["rng3"], cfg["sh1"]

  def body(src_hbm, dst_hbm, out_hbm, srcc, dstc, b0, b1, b2, b3):
    w = _wid()
    iota = lax.iota(jnp.int32, L)
    padv = jnp.full((L,), jnp.int32((1 << sh1) - 1), jnp.int32)

    e0 = pl.multiple_of(w * eps, 8)
    pltpu.sync_copy(src_hbm.at[pl.ds(e0, eps)], srcc.at[pl.ds(0, eps)])
    pltpu.sync_copy(dst_hbm.at[pl.ds(e0, eps)], dstc.at[pl.ds(0, eps)])
    rem = eps % L
    t0 = eps - rem
    if rem:
      dv = dstc[pl.ds(t0, L)]
      dstc[pl.ds(t0, L)] = jnp.where(iota < rem, dv,
                                     jnp.int32(1 << 30))
      sv = srcc[pl.ds(t0, L)]
      srcc[pl.ds(t0, L)] = jnp.where(iota < rem, sv, 0)
      t0 += L
    for t in range(t0 // L, epsp // L):
      srcc[pl.ds(t * L, L)] = jnp.zeros((L,), jnp.int32)
      dstc[pl.ds(t * L, L)] = jnp.full((L,), jnp.int32(1 << 30), jnp.int32)

    bufs = (b0, b1, b2, b3)
    sizes = (rng, rng, rng, rng3)

    def _grp(t, carry):
      states = [tuple(carry[4 * r:4 * r + 4]) for r in range(NRG)]
      dv = dstc[pl.ds(t * L, L)]
      sv = srcc[pl.ds(t * L, L)]
      out = []
      for r in range(NRG):
        lb = dv - r * rng
        oki = 1 - lax.shift_right_logical(lb | (sizes[r] - 1 - lb), 31)
        vals = (sv << sh1) | (lb * oki)
        st = _insert_lanes(bufs[r], 16, iota, states[r], vals, oki)
        _store_blocks(bufs[r], 16, st)
        out.extend(st)
      return tuple(out)

    zv = jnp.zeros((L,), jnp.int32)
    init = ()
    for _ in range(NRG):
      init = init + (zv, zv, jnp.int32(0), jnp.int32(0))
    carry = lax.fori_loop(0, epsp // L, _grp, init)
    for r in range(NRG):
      st = tuple(carry[4 * r:4 * r + 4])
      _store_blocks(bufs[r], 16, st)
      _finish_bucket(bufs[r], 16, iota, st[0], st[2], padv, 0)
      pltpu.sync_copy(bufs[r], out_hbm.at[w, r])

  return body


def _bin1(src, dst, cfg=CFG, *, interpret=False):
  f = pl.kernel(
      _make_bin1(cfg),
      out_type=jax.ShapeDtypeStruct((NW, NRG, cfg["cap1"]), jnp.int32),
      mesh=_mesh(),
      scratch_types=[
          pltpu.VMEM((cfg["epsp"],), jnp.int32),
          pltpu.VMEM((cfg["epsp"],), jnp.int32),
          pltpu.VMEM((cfg["cap1"],), jnp.int32),
          pltpu.VMEM((cfg["cap1"],), jnp.int32),
          pltpu.VMEM((cfg["cap1"],), jnp.int32),
          pltpu.VMEM((cfg["cap1"],), jnp.int32),
      ],
      interpret=interpret,
  )
  return f(src, dst)


def _make_bin2(cfg):
  rpw, last, rng, sh1, sh2, bk = (cfg["rpw"], cfg["last"], cfg["rng"],
                                  cfg["sh1"], cfg["sh2"], cfg["bk"])
  dummy = cfg["dummy"]

  def body(slots_hbm, out_hbm, segbuf, bucket):
    w = _wid()
    iota = lax.iota(jnp.int32, L)
    padv = jnp.full((L,), jnp.int32(dummy), jnp.int32)
    r = w >> 3
    rbase = r * rng
    base = pl.multiple_of(w * rpw, 8)
    bound = jnp.where(w == NW - 1, last, rpw)
    mask1 = (1 << sh1) - 1

    def _seg(i, _):
      pltpu.sync_copy(slots_hbm.at[i, r], segbuf)
      cnt = segbuf[pl.ds(0, L)][0]

      def _grp(t, carry):
        p = segbuf[pl.ds(16 + t * L, L)]
        gdst = rbase + (p & mask1)
        lv = gdst - base
        oki = 1 - lax.shift_right_logical(lv | (bound - 1 - lv), 31)
        src = lax.shift_right_logical(p, sh1)
        vals = (src << sh2) | (lv * oki + jnp.int32(dummy) * (1 - oki))
        st = _insert_lanes(bucket, 16, iota, tuple(carry), vals, oki)
        _store_blocks(bucket, 16, st)
        return st

      zv = jnp.zeros((L,), jnp.int32)
      st = lax.fori_loop(0, (cnt + L - 1) // L, _grp,
                         (zv, zv, jnp.int32(0), jnp.int32(0)))
      _store_blocks(bucket, 16, st)
      _finish_bucket(bucket, 16, iota, st[0], st[2], padv,
                     4 * bk // L - 1)
      pltpu.sync_copy(bucket, out_hbm.at[w, i])
      return 0
    lax.fori_loop(0, NW, _seg, 0)

  return body


def _bin2(slots, cfg=CFG, *, interpret=False):
  f = pl.kernel(
      _make_bin2(cfg),
      out_type=jax.ShapeDtypeStruct((NW, NW, cfg["cap2"]), jnp.int32),
      mesh=_mesh(),
      scratch_types=[
          pltpu.VMEM((cfg["cap1"],), jnp.int32),
          pltpu.VMEM((cfg["cap2"],), jnp.int32),
      ],
      interpret=interpret,
  )
  return f(slots)


def _make_acc(cfg, dh):
  bk = cfg["bk"]
  rpw, last, accr, sh2 = cfg["rpw"], cfg["last"], cfg["accr"], cfg["sh2"]
  ncg = dh // L
  mask2 = (1 << sh2) - 1

  def body(xh_hbm, slots_hbm, out_hbm, segbuf,
           sidx0, sidx1, sidx2, sidx3, rows0, rows1, rows2, rows3,
           acc, cnt, sem0, sem1, sem2, sem3):
    w = _wid()
    base = pl.multiple_of(w * rpw, 8)
    zf = jnp.zeros((L,), jnp.float32)
    onef = jnp.ones((L,), jnp.float32)
    sidxs = (sidx0, sidx1, sidx2, sidx3)
    rowss = (rows0, rows1, rows2, rows3)
    sems = (sem0, sem1, sem2, sem3)

    def _zr(rr, _):
      for g in range(ncg):
        acc[rr, pl.ds(g * L, L)] = zf
      cnt[rr, :] = zf
      return 0
    lax.fori_loop(0, accr, _zr, 0)

    def _seg(i, _):
      pltpu.sync_copy(slots_hbm.at[w, i], segbuf)
      ecnt = segbuf[pl.ds(0, L)][0]

      def _blk(b, _):
        b0 = 16 + b * bk
        for u in range(bk // L):
          p = segbuf[pl.ds(b0 + u * L, L)]
          sidx0[pl.ds(u * L, L)] = lax.shift_right_logical(p, sh2)
        pltpu.async_copy(xh_hbm.at[sidx0], rows0, sem0).wait()

        def _rmw(g, _):
          p = segbuf[pl.ds(b0 + g * L, L)]
          locs = p & mask2
          for k in range(L):
            rr = locs[k]
            sr = g * L + k
            for gc in range(ncg):
              plsc.addupdate(acc.at[rr, pl.ds(gc * L, L)],
                             rows0[sr, pl.ds(gc * L, L)])
            plsc.addupdate(cnt.at[rr, :], onef)
          return 0
        lax.fori_loop(0, bk // L, _rmw, 0)
        return 0
      lax.fori_loop(0, (ecnt + bk - 1) // bk, _blk, 0)
      return 0
    lax.fori_loop(0, NW, _seg, 0)

    def _div(rr, _):
      inv = 1.0 / jnp.maximum(cnt[rr, :], 1.0)
      for gc in range(ncg):
        acc[rr, pl.ds(gc * L, L)] = acc[rr, pl.ds(gc * L, L)] * inv
      return 0
    lax.fori_loop(0, last, _div, 0)

    @pl.when(w < NW - 1)
    def _():
      pltpu.sync_copy(acc.at[pl.ds(0, rpw)], out_hbm.at[pl.ds(base, rpw)])
    @pl.when(w == NW - 1)
    def _():
      pltpu.sync_copy(acc.at[pl.ds(0, last)], out_hbm.at[pl.ds(base, last)])

  return body


def _acc_half(xh, slots, cfg=CFG, *, interpret=False):
  dh = xh.shape[1]
  f = pl.kernel(
      _make_acc(cfg, dh),
      out_type=jax.ShapeDtypeStruct((cfg["n"], dh), jnp.float32),
      mesh=_mesh(),
      scratch_types=[
          pltpu.VMEM((cfg["cap2"],), jnp.int32),
          pltpu.VMEM((cfg["bk"],), jnp.int32),
          pltpu.VMEM((cfg["bk"],), jnp.int32),
          pltpu.VMEM((cfg["bk"],), jnp.int32),
          pltpu.VMEM((cfg["bk"],), jnp.int32),
          pltpu.VMEM((cfg["bk"], dh), jnp.float32),
          pltpu.VMEM((cfg["bk"], dh), jnp.float32),
          pltpu.VMEM((cfg["bk"], dh), jnp.float32),
          pltpu.VMEM((cfg["bk"], dh), jnp.float32),
          pltpu.VMEM((cfg["accr"], dh), jnp.float32),
          pltpu.VMEM((cfg["accr"], L), jnp.float32),
          pltpu.SemaphoreType.DMA,
          pltpu.SemaphoreType.DMA,
          pltpu.SemaphoreType.DMA,
          pltpu.SemaphoreType.DMA,
      ],
      interpret=interpret,
  )
  return f(xh, slots)


def _segment_mean_halves(x, src, dst, cfg=CFG, *, interpret=False):
  dh = cfg["d"] // 2
  slots1 = _bin1(src, dst, cfg, interpret=interpret)
  slots2 = _bin2(slots1, cfg, interpret=interpret)
  m0 = _acc_half(x[:, :dh], slots2, cfg, interpret=interpret)
  m1 = _acc_half(x[:, dh:], slots2, cfg, interpret=interpret)
  return m0, m1


def _tc_body(m0_ref, m1_ref, x_ref, wl0_ref, wl1_ref, bl_ref, wr_ref,
             wlin_ref, blin_ref, out_ref):
  h = jnp.dot(m0_ref[...], wl0_ref[...], preferred_element_type=jnp.float32)
  h = h + jnp.dot(m1_ref[...], wl1_ref[...],
                  preferred_element_type=jnp.float32)
  h = h + bl_ref[...]
  h = h + jnp.dot(x_ref[...], wr_ref[...], preferred_element_type=jnp.float32)
  h = jnp.maximum(h, 0.0)
  out_ref[...] = (
      jnp.dot(h, wlin_ref[...], preferred_element_type=jnp.float32)
      + blin_ref[...])


def _dense(m0, m1, x, W_l, b_l, W_r, W_lin, b_lin, *, interpret=False):
  rows = 1000
  dh = D // 2
  grid = (N // rows,)
  half_spec = pl.BlockSpec((rows, dh), lambda i: (i, 0))
  row_spec = pl.BlockSpec((rows, D), lambda i: (i, 0))
  w_spec = pl.BlockSpec((D, D), lambda i: (0, 0))
  wh_spec = pl.BlockSpec((dh, D), lambda i: (0, 0))
  b_spec = pl.BlockSpec((1, D), lambda i: (0, 0))
  return pl.pallas_call(
      _tc_body,
      grid=grid,
      in_specs=[half_spec, half_spec, row_spec, wh_spec, wh_spec, b_spec,
                w_spec, w_spec, b_spec],
      out_specs=row_spec,
      out_shape=jax.ShapeDtypeStruct((N, D), jnp.float32),
      interpret=interpret,
  )(m0, m1, x, W_l[:dh], W_l[dh:], b_l.reshape(1, D), W_r, W_lin,
    b_lin.reshape(1, D))


@jax.jit
def kernel(x, edge_index, W_l, b_l, W_r, W_lin, b_lin):
  src = edge_index[0].astype(jnp.int32)
  dst = edge_index[1].astype(jnp.int32)
  m0, m1 = _segment_mean_halves(x, src, dst)
  return _dense(m0, m1, x, W_l, b_l, W_r, W_lin, b_lin)
